# baseline (speedup 1.0000x reference)
import functools

import jax
import jax.numpy as jnp
from jax import lax
from jax.experimental import pallas as pl
from jax.experimental.pallas import tpu as pltpu
from jax.experimental.pallas import tpu_sc as plsc

GRID = 21
SIZE_2D = GRID * GRID
R = 8
B = 8
N = 500000
NW = 32
RSTRIDE = 448
BR = B * R * RSTRIDE
TILE = 128
TW = 122
WCOLS = TW * TILE
CW = 4096
CW_LAST = WCOLS - 3 * CW
REM0 = NW * WCOLS
TAIL0 = REM0 + 2 * TILE
TAILC = N - TAIL0


def _hist_body(pcd_hbm, tail_hbm, coef_hbm, out_hbm, xbuf, ybuf, hist, coefv,
               tbuf, dsem):
    c = lax.axis_index("c")
    s = lax.axis_index("s")
    w = c * 16 + s
    cstart_w = w * WCOLS

    pltpu.sync_copy(coef_hbm, coefv)

    zeros = jnp.zeros((16,), jnp.int32)
    ones = jnp.ones((16,), jnp.int32)

    def zbody(i, _):
        hist[pl.ds(i * 16, 16)] = zeros
        return 0
    lax.fori_loop(0, BR // 16, zbody, 0)

    a0x = coefv[pl.ds(0, 16)]
    b0x = coefv[pl.ds(16, 16)]
    a0y = coefv[pl.ds(32, 16)]
    b0y = coefv[pl.ds(48, 16)]
    a1x = coefv[pl.ds(64, 16)]
    b1x = coefv[pl.ds(80, 16)]
    a1y = coefv[pl.ds(96, 16)]
    b1y = coefv[pl.ds(112, 16)]
    cxv = coefv[pl.ds(128, 16)]
    cyv = coefv[pl.ds(144, 16)]

    def do_group(xv, yv, boffs):
        u0 = xv * a0x + yv * b0x
        v0 = xv * a0y + yv * b0y
        u1 = xv * a1x + yv * b1x
        v1 = xv * a1y + yv * b1y
        pa0 = (cxv + u0).astype(jnp.int32)
        pb0 = (cxv + v0).astype(jnp.int32)
        pc0 = (cxv - u0).astype(jnp.int32)
        pd0 = (cxv - v0).astype(jnp.int32)
        pa1 = (cxv + u1).astype(jnp.int32)
        pb1 = (cxv + v1).astype(jnp.int32)
        pc1 = (cxv - u1).astype(jnp.int32)
        pd1 = (cxv - v1).astype(jnp.int32)
        pairs = (
            (pa0, pb0), (pa1, pb1),
            (pd0, pa0), (pd1, pa1),
            (pc0, pd0), (pc1, pd1),
            (pb0, pc0), (pb1, pc1),
        )
        for r, (px, py) in enumerate(pairs):
            plsc.addupdate_scatter(
                hist.at[pl.ds(boffs + r * RSTRIDE, RSTRIDE)],
                [px * GRID + py], ones)

    def process_block(cstart, cols, sem):
        cpx = pltpu.async_copy(pcd_hbm.at[0, :, pl.ds(cstart, cols)],
                               xbuf.at[:, pl.ds(0, cols)], sem)
        cpy = pltpu.async_copy(pcd_hbm.at[1, :, pl.ds(cstart, cols)],
                               ybuf.at[:, pl.ds(0, cols)], sem)
        cpx.wait()
        cpy.wait()

        def rbody(row, _):
            boffs = row * (R * RSTRIDE)

            @plsc.parallel_loop(0, cols // 16, unroll=8)
            def _(g):
                xv = xbuf[row, pl.ds(g * 16, 16)]
                yv = ybuf[row, pl.ds(g * 16, 16)]
                do_group(xv, yv, boffs)
            return 0
        lax.fori_loop(0, B, rbody, 0)

    def chunk_body(i, _):
        process_block(cstart_w + i * CW, CW, dsem)
        return 0
    lax.fori_loop(0, 3, chunk_body, 0)
    process_block(cstart_w + 3 * CW, CW_LAST, dsem)

    @pl.when(w == 0)
    def _():
        process_block(REM0, TILE, dsem)

    @pl.when(w == 1)
    def _():
        process_block(REM0 + TILE, TILE, dsem)

    @pl.when(w == 2)
    def _():
        pltpu.sync_copy(tail_hbm, tbuf)

        def trbody(row, _):
            boffs = row * (R * RSTRIDE)
            for g in range(TAILC // 16):
                xv = tbuf[pl.ds(row * TAILC + g * 16, 16)]
                yv = tbuf[pl.ds(B * TAILC + row * TAILC + g * 16, 16)]
                do_group(xv, yv, boffs)
            return 0
        lax.fori_loop(0, B, trbody, 0)

    pltpu.sync_copy(hist, out_hbm.at[pl.ds(w * BR, BR)])


_hist_call = functools.partial(
    pl.kernel,
    mesh=plsc.VectorSubcoreMesh(core_axis_name="c", subcore_axis_name="s"),
    out_type=jax.ShapeDtypeStruct((NW * BR,), jnp.int32),
    scratch_types=[
        pltpu.VMEM((B, CW), jnp.float32),
        pltpu.VMEM((B, CW), jnp.float32),
        pltpu.VMEM((BR,), jnp.int32),
        pltpu.VMEM((10 * 16,), jnp.float32),
        pltpu.VMEM((2 * B * TAILC,), jnp.float32),
        pltpu.SemaphoreType.DMA,
    ],
    compiler_params=pltpu.CompilerParams(needs_layout_passes=False),
)(_hist_body)


@jax.jit
def kernel(pcd, mats, offset_2d, voxel_size_2d):
    inv = (1.0 / voxel_size_2d).astype(jnp.float32)
    coef = jnp.stack([
        mats[0, 0, 0] * inv[0], mats[0, 0, 1] * inv[0],
        mats[0, 1, 0] * inv[1], mats[0, 1, 1] * inv[1],
        mats[1, 0, 0] * inv[0], mats[1, 0, 1] * inv[0],
        mats[1, 1, 0] * inv[1], mats[1, 1, 1] * inv[1],
        offset_2d[0] * inv[0], offset_2d[1] * inv[1],
    ])
    coef16 = jnp.broadcast_to(
        coef[:, None], (10, 16)).reshape(-1).astype(jnp.float32)

    pcd_t = jnp.transpose(pcd, (2, 0, 1))
    tailxy = jnp.transpose(pcd[:, TAIL0:, :2], (2, 0, 1)).reshape(-1)

    out = _hist_call(pcd_t, tailxy, coef16)

    part = out.reshape(NW, B, R, RSTRIDE)[..., :SIZE_2D].sum(axis=0).astype(jnp.float32)
    feat = part.transpose(0, 2, 1) / jnp.float32(N)
    return feat


# speedup vs baseline: 1.2952x; 1.2952x over previous
import functools

import jax
import jax.numpy as jnp
from jax import lax
from jax.experimental import pallas as pl
from jax.experimental.pallas import tpu as pltpu
from jax.experimental.pallas import tpu_sc as plsc

GRID = 21
SIZE_2D = GRID * GRID
R = 8
B = 8
N = 500000
NW = 32
RSTRIDE = 448
BR = B * R * RSTRIDE
TILE = 128
TW = 122
WCOLS = TW * TILE
CW = 4096
CW_LAST = WCOLS - 3 * CW
REM0 = NW * WCOLS
TAIL0 = REM0 + 2 * TILE
TAILC = N - TAIL0


def _hist_body(pcd_hbm, tail_hbm, coef_hbm, out_hbm, xbuf, ybuf, hist, coefv,
               tbuf, dsem):
    c = lax.axis_index("c")
    s = lax.axis_index("s")
    w = c * 16 + s
    cstart_w = w * WCOLS

    pltpu.sync_copy(coef_hbm, coefv)

    zeros = jnp.zeros((16,), jnp.int32)
    ones = jnp.ones((16,), jnp.int32)

    def zbody(i, _):
        hist[pl.ds(i * 16, 16)] = zeros
        return 0
    lax.fori_loop(0, BR // 16, zbody, 0)

    a0x = coefv[pl.ds(0, 16)]
    b0x = coefv[pl.ds(16, 16)]
    a0y = coefv[pl.ds(32, 16)]
    b0y = coefv[pl.ds(48, 16)]
    a1x = coefv[pl.ds(64, 16)]
    b1x = coefv[pl.ds(80, 16)]
    a1y = coefv[pl.ds(96, 16)]
    b1y = coefv[pl.ds(112, 16)]
    cxv = coefv[pl.ds(128, 16)]
    cyv = coefv[pl.ds(144, 16)]

    def do_group(xv, yv, boffs):
        u0 = xv * a0x + yv * b0x
        v0 = xv * a0y + yv * b0y
        u1 = xv * a1x + yv * b1x
        v1 = xv * a1y + yv * b1y
        pa0 = (cxv + u0).astype(jnp.int32)
        pb0 = (cxv + v0).astype(jnp.int32)
        pc0 = (cxv - u0).astype(jnp.int32)
        pd0 = (cxv - v0).astype(jnp.int32)
        pa1 = (cxv + u1).astype(jnp.int32)
        pb1 = (cxv + v1).astype(jnp.int32)
        pc1 = (cxv - u1).astype(jnp.int32)
        pd1 = (cxv - v1).astype(jnp.int32)
        pairs = (
            (pa0, pb0), (pa1, pb1),
            (pd0, pa0), (pd1, pa1),
            (pc0, pd0), (pc1, pd1),
            (pb0, pc0), (pb1, pc1),
        )
        for r, (px, py) in enumerate(pairs):
            plsc.addupdate_scatter(
                hist.at[pl.ds(boffs + r * RSTRIDE, RSTRIDE)],
                [px * GRID + py], ones)

    def process_block(cstart, cols, sem, ur=16):
        cpx = pltpu.async_copy(pcd_hbm.at[0, :, pl.ds(cstart, cols)],
                               xbuf.at[:, pl.ds(0, cols)], sem)
        cpy = pltpu.async_copy(pcd_hbm.at[1, :, pl.ds(cstart, cols)],
                               ybuf.at[:, pl.ds(0, cols)], sem)
        cpx.wait()
        cpy.wait()

        def rbody(row, _):
            boffs = row * (R * RSTRIDE)

            def gbody(gu, _):
                base = gu * (16 * ur)
                for j in range(ur):
                    xv = xbuf[row, pl.ds(base + j * 16, 16)]
                    yv = ybuf[row, pl.ds(base + j * 16, 16)]
                    do_group(xv, yv, boffs)
                return 0
            lax.fori_loop(0, cols // (16 * ur), gbody, 0)
            return 0
        lax.fori_loop(0, B, rbody, 0)

    def chunk_body(i, _):
        process_block(cstart_w + i * CW, CW, dsem)
        return 0
    lax.fori_loop(0, 3, chunk_body, 0)
    process_block(cstart_w + 3 * CW, CW_LAST, dsem)

    @pl.when(w == 0)
    def _():
        process_block(REM0, TILE, dsem, ur=8)

    @pl.when(w == 1)
    def _():
        process_block(REM0 + TILE, TILE, dsem, ur=8)

    @pl.when(w == 2)
    def _():
        pltpu.sync_copy(tail_hbm, tbuf)

        def trbody(row, _):
            boffs = row * (R * RSTRIDE)
            for g in range(TAILC // 16):
                xv = tbuf[pl.ds(row * TAILC + g * 16, 16)]
                yv = tbuf[pl.ds(B * TAILC + row * TAILC + g * 16, 16)]
                do_group(xv, yv, boffs)
            return 0
        lax.fori_loop(0, B, trbody, 0)

    pltpu.sync_copy(hist, out_hbm.at[pl.ds(w * BR, BR)])


_hist_call = functools.partial(
    pl.kernel,
    mesh=plsc.VectorSubcoreMesh(core_axis_name="c", subcore_axis_name="s"),
    out_type=jax.ShapeDtypeStruct((NW * BR,), jnp.int32),
    scratch_types=[
        pltpu.VMEM((B, CW), jnp.float32),
        pltpu.VMEM((B, CW), jnp.float32),
        pltpu.VMEM((BR,), jnp.int32),
        pltpu.VMEM((10 * 16,), jnp.float32),
        pltpu.VMEM((2 * B * TAILC,), jnp.float32),
        pltpu.SemaphoreType.DMA,
    ],
    compiler_params=pltpu.CompilerParams(needs_layout_passes=False),
)(_hist_body)


@jax.jit
def kernel(pcd, mats, offset_2d, voxel_size_2d):
    inv = (1.0 / voxel_size_2d).astype(jnp.float32)
    coef = jnp.stack([
        mats[0, 0, 0] * inv[0], mats[0, 0, 1] * inv[0],
        mats[0, 1, 0] * inv[1], mats[0, 1, 1] * inv[1],
        mats[1, 0, 0] * inv[0], mats[1, 0, 1] * inv[0],
        mats[1, 1, 0] * inv[1], mats[1, 1, 1] * inv[1],
        offset_2d[0] * inv[0], offset_2d[1] * inv[1],
    ])
    coef16 = jnp.broadcast_to(
        coef[:, None], (10, 16)).reshape(-1).astype(jnp.float32)

    pcd_t = jnp.transpose(pcd, (2, 0, 1))
    tailxy = jnp.transpose(pcd[:, TAIL0:, :2], (2, 0, 1)).reshape(-1)

    out = _hist_call(pcd_t, tailxy, coef16)

    part = out.reshape(NW, B, R, RSTRIDE)[..., :SIZE_2D].sum(axis=0).astype(jnp.float32)
    feat = part.transpose(0, 2, 1) / jnp.float32(N)
    return feat
